# baseline (speedup 1.0000x reference)
import functools

import jax
import jax.numpy as jnp
from jax import lax
from jax.experimental import pallas as pl
from jax.experimental.pallas import tpu as pltpu
from jax.experimental.pallas import tpu_sc as plsc

_B = 1024
_S = 50
_K = 100000
_TOPK = 30
_D = 128
_EPS = 1e-12

_ROWS = 8
_CH = 1280
_NCH = 80
_KP = _CH * _NCH
_NEG = float("-inf")


def _topk_body(x_ref, lab_ref, s_ref):
    s_ref[:, _K:] = jnp.full((_ROWS, _KP - _K), _NEG, jnp.float32)
    s_ref[:, : _K] = x_ref[...]

    lane80 = lax.broadcasted_iota(jnp.int32, (_ROWS, _NCH), 1)
    row80 = lax.broadcasted_iota(jnp.int32, (_ROWS, _NCH), 0)
    lane32 = lax.broadcasted_iota(jnp.int32, (_ROWS, 32), 1)
    row32 = lax.broadcasted_iota(jnp.int32, (_ROWS, 32), 0)
    lane_ch = lax.broadcasted_iota(jnp.int32, (1, _CH), 1)
    lane1d = lax.broadcasted_iota(jnp.int32, (_NCH,), 0)

    def build(c, l1):
        start = pl.multiple_of(c * _CH, 128)
        cm = jnp.max(s_ref[:, pl.ds(start, _CH)], axis=1, keepdims=True)
        return jnp.where(lane80 == c, cm, l1)

    l1 = lax.fori_loop(
        0, _NCH, build, jnp.full((_ROWS, _NCH), _NEG, jnp.float32)
    )

    def body(t, carry):
        l1, lab = carry
        m = jnp.max(l1, axis=1, keepdims=True)
        cvec = jnp.min(jnp.where(l1 == m, lane80, _NCH), axis=1, keepdims=True)
        upd = []
        for r in range(_ROWS):
            m_r = m[r, 0]
            c_r = cvec[r, 0]
            start = pl.multiple_of(c_r * _CH, 128)
            chunk = s_ref[pl.ds(r, 1), pl.ds(start, _CH)]
            lidx = jnp.min(jnp.where(chunk == m_r, lane_ch, _CH))
            zapped = jnp.where(lane_ch == lidx, _NEG, chunk)
            s_ref[pl.ds(r, 1), pl.ds(start, _CH)] = zapped
            upd.append((c_r, jnp.max(zapped), c_r * _CH + lidx))
        for r in range(_ROWS):
            c_r, nm, glob = upd[r]
            l1 = jnp.where((row80 == r) & (lane80 == c_r), nm, l1)
            lab = jnp.where((row32 == r) & (lane32 == t), glob, lab)
        return l1, lab

    _, lab = lax.fori_loop(
        0, _TOPK, body, (l1, jnp.zeros((_ROWS, 32), jnp.int32))
    )
    lab_ref[...] = lab


def _topk(preds_attr):
    grid = _B // _ROWS
    lab = pl.pallas_call(
        _topk_body,
        grid=(grid,),
        in_specs=[pl.BlockSpec((_ROWS, _K), lambda i: (i, 0))],
        out_specs=pl.BlockSpec((_ROWS, 32), lambda i: (i, 0)),
        out_shape=jax.ShapeDtypeStruct((_B, 32), jnp.int32),
        scratch_shapes=[
            pltpu.VMEM((_ROWS, _KP), jnp.float32),
        ],
        compiler_params=pltpu.CompilerParams(
            dimension_semantics=("arbitrary",),
        ),
    )(preds_attr)
    return lab[:, :_TOPK]


def _make_sc_gather():
    nc, ns = 2, 16
    nw = nc * ns
    n = _B * _TOPK
    b_per_w = n // nw
    mesh = plsc.VectorSubcoreMesh(core_axis_name="c", subcore_axis_name="s")

    @functools.partial(
        pl.kernel,
        mesh=mesh,
        out_type=jax.ShapeDtypeStruct((n, _D), jnp.float32),
        scratch_types=[
            pltpu.VMEM((b_per_w,), jnp.int32),
            pltpu.VMEM((b_per_w, _D), jnp.float32),
            pltpu.SemaphoreType.DMA,
        ],
    )
    def gather_k(table_hbm, idx_hbm, out_hbm, idx_v, rows_v, sem):
        wid = lax.axis_index("s") * nc + lax.axis_index("c")
        base = wid * b_per_w
        pltpu.sync_copy(idx_hbm.at[pl.ds(base, b_per_w)], idx_v)
        pltpu.async_copy(table_hbm.at[idx_v], rows_v, sem).wait()
        pltpu.sync_copy(rows_v, out_hbm.at[pl.ds(base, b_per_w)])

    return gather_k


def _ln_body(x_ref, pos_ref, g_ref, b_ref, o_ref):
    x = x_ref[...]
    pos = jnp.tile(pos_ref[...], (x.shape[0] // _TOPK, 1))
    y = x + pos
    mu = jnp.mean(y, axis=1, keepdims=True)
    d = y - mu
    var = jnp.mean(d * d, axis=1, keepdims=True)
    o_ref[...] = d / jnp.sqrt(var + _EPS) * g_ref[...] + b_ref[...]


def _ln(embs_flat, pos_emb, ln_gamma, ln_beta):
    rows = 240
    grid = (_B * _TOPK) // rows
    return pl.pallas_call(
        _ln_body,
        grid=(grid,),
        in_specs=[
            pl.BlockSpec((rows, _D), lambda i: (i, 0)),
            pl.BlockSpec((_TOPK, _D), lambda i: (0, 0)),
            pl.BlockSpec((1, _D), lambda i: (0, 0)),
            pl.BlockSpec((1, _D), lambda i: (0, 0)),
        ],
        out_specs=pl.BlockSpec((rows, _D), lambda i: (i, 0)),
        out_shape=jax.ShapeDtypeStruct((_B * _TOPK, _D), jnp.float32),
        compiler_params=pltpu.CompilerParams(
            dimension_semantics=("arbitrary",),
        ),
    )(embs_flat, pos_emb, ln_gamma, ln_beta)


def kernel(encoder_hidden_states, preds_attr, word_emb, pos_emb, ln_gamma, ln_beta):
    labels = _topk(preds_attr)
    idx_flat = labels.reshape(_B * _TOPK)
    embs_flat = _make_sc_gather()(word_emb, idx_flat)
    out = _ln(
        embs_flat,
        pos_emb,
        ln_gamma.reshape(1, _D),
        ln_beta.reshape(1, _D),
    )
    return out.reshape(_B, _TOPK, _D), labels


# speedup vs baseline: 2.3076x; 2.3076x over previous
import functools

import jax
import jax.numpy as jnp
from jax import lax
from jax.experimental import pallas as pl
from jax.experimental.pallas import tpu as pltpu
from jax.experimental.pallas import tpu_sc as plsc

_B = 1024
_K = 100000
_TOPK = 30
_D = 128
_EPS = 1e-12

_NC, _NS, _L = 2, 16, 16
_NW = _NC * _NS
_RPW = _B // _NW
_NV = _K // _L
_UNROLL = 10
_CAP = 4096
_NEG = float("-inf")
_IMAX = 2**31 - 1


def _extract30(val_at, idx_at, nv, zap, iota):
    lab_a = jnp.zeros((_L,), jnp.int32)
    lab_b = jnp.zeros((_L,), jnp.int32)
    for k in range(_TOPK):
        def scan(j, mp):
            m, p = mp
            v = val_at(j)
            ix = idx_at(j)
            gt = v > m
            return jnp.where(gt, v, m), jnp.where(gt, ix, p)

        m, p = lax.fori_loop(
            0, nv, scan,
            (jnp.full((_L,), _NEG, jnp.float32), jnp.zeros((_L,), jnp.int32)),
        )
        mm = jnp.max(m)
        psel = jnp.min(jnp.where(m == mm, p, _IMAX))

        def dozap(j, _):
            v = val_at(j)
            ix = idx_at(j)
            zap(j, (v == mm) & (ix == psel))
            return 0

        lax.fori_loop(0, nv, dozap, 0)
        if k < _L:
            lab_a = jnp.where(iota == k, psel, lab_a)
        else:
            lab_b = jnp.where(iota == (k - _L), psel, lab_b)
    return lab_a, lab_b


def _make_sc_topk():
    mesh = plsc.VectorSubcoreMesh(core_axis_name="c", subcore_axis_name="s")

    @functools.partial(
        pl.kernel,
        mesh=mesh,
        out_type=jax.ShapeDtypeStruct((_B, 32), jnp.int32),
        scratch_types=[
            pltpu.VMEM((_K,), jnp.float32),
            pltpu.VMEM((_CAP + 16,), jnp.float32),
            pltpu.VMEM((_CAP + 16,), jnp.int32),
            pltpu.VMEM((32,), jnp.int32),
        ],
        compiler_params=pltpu.CompilerParams(needs_layout_passes=False),
    )
    def topk_k(preds_hbm, out_hbm, row_v, candv, candi, lab_v):
        wid = lax.axis_index("s") * _NC + lax.axis_index("c")
        iota = lax.broadcasted_iota(jnp.int32, (_L,), 0)

        def row_body(i, _):
            r = wid * _RPW + i
            pltpu.sync_copy(preds_hbm.at[r], row_v)

            def pass_a(jo, m12):
                m1, m2 = m12
                for u in range(_UNROLL):
                    v = row_v[pl.ds((jo * _UNROLL + u) * _L, _L)]
                    gt1 = v > m1
                    m2 = jnp.where(gt1, m1, jnp.maximum(m2, v))
                    m1 = jnp.maximum(m1, v)
                return m1, m2

            neg = jnp.full((_L,), _NEG, jnp.float32)
            _, m2 = lax.fori_loop(0, _NV // _UNROLL, pass_a, (neg, neg))
            t = jnp.min(m2)
            t_vec = jnp.full((_L,), t, jnp.float32)

            def pass_b(jo, cnt):
                for u in range(_UNROLL):
                    j = jo * _UNROLL + u
                    v = row_v[pl.ds(j * _L, _L)]
                    mask = v >= t_vec

                    def append(c):
                        base = jnp.minimum(c, _CAP)
                        plsc.store_compressed(
                            candv.at[pl.ds(base, _L)], v, mask=mask)
                        plsc.store_compressed(
                            candi.at[pl.ds(base, _L)],
                            iota + j * _L, mask=mask)
                        return c + jnp.sum(mask.astype(jnp.int32))

                    cnt = lax.cond(jnp.any(mask), append, lambda c: c, cnt)
                return cnt

            cnt = lax.fori_loop(0, _NV // _UNROLL, pass_b, jnp.int32(0))
            candv[pl.ds(jnp.minimum(cnt, _CAP), _L)] = neg

            def fast(_):
                nv = (cnt + _L - 1) // _L

                def zapc(j, hit):
                    v = candv[pl.ds(j * _L, _L)]
                    candv[pl.ds(j * _L, _L)] = jnp.where(hit, _NEG, v)

                return _extract30(
                    lambda j: candv[pl.ds(j * _L, _L)],
                    lambda j: candi[pl.ds(j * _L, _L)],
                    nv, zapc, iota)

            def slow(_):
                def zapr(j, hit):
                    v = row_v[pl.ds(j * _L, _L)]
                    row_v[pl.ds(j * _L, _L)] = jnp.where(hit, _NEG, v)

                return _extract30(
                    lambda j: row_v[pl.ds(j * _L, _L)],
                    lambda j: iota + j * _L,
                    _NV, zapr, iota)

            lab_a, lab_b = lax.cond(cnt <= _CAP, fast, slow, 0)
            lab_v[pl.ds(0, _L)] = lab_a
            lab_v[pl.ds(_L, _L)] = lab_b
            pltpu.sync_copy(lab_v, out_hbm.at[r])
            return 0

        lax.fori_loop(0, _RPW, row_body, 0)

    return topk_k


def _make_sc_gather():
    n = _B * _TOPK
    b_per_w = n // _NW
    mesh = plsc.VectorSubcoreMesh(core_axis_name="c", subcore_axis_name="s")

    @functools.partial(
        pl.kernel,
        mesh=mesh,
        out_type=jax.ShapeDtypeStruct((n, _D), jnp.float32),
        scratch_types=[
            pltpu.VMEM((b_per_w,), jnp.int32),
            pltpu.VMEM((b_per_w, _D), jnp.float32),
            pltpu.SemaphoreType.DMA,
        ],
    )
    def gather_k(table_hbm, idx_hbm, out_hbm, idx_v, rows_v, sem):
        wid = lax.axis_index("s") * _NC + lax.axis_index("c")
        base = wid * b_per_w
        pltpu.sync_copy(idx_hbm.at[pl.ds(base, b_per_w)], idx_v)
        pltpu.async_copy(table_hbm.at[idx_v], rows_v, sem).wait()
        pltpu.sync_copy(rows_v, out_hbm.at[pl.ds(base, b_per_w)])

    return gather_k


def _ln_body(x_ref, pos_ref, g_ref, b_ref, o_ref):
    x = x_ref[...]
    pos = jnp.tile(pos_ref[...], (x.shape[0] // _TOPK, 1))
    y = x + pos
    mu = jnp.mean(y, axis=1, keepdims=True)
    d = y - mu
    var = jnp.mean(d * d, axis=1, keepdims=True)
    o_ref[...] = d / jnp.sqrt(var + _EPS) * g_ref[...] + b_ref[...]


def _ln(embs_flat, pos_emb, ln_gamma, ln_beta):
    rows = 240
    grid = (_B * _TOPK) // rows
    return pl.pallas_call(
        _ln_body,
        grid=(grid,),
        in_specs=[
            pl.BlockSpec((rows, _D), lambda i: (i, 0)),
            pl.BlockSpec((_TOPK, _D), lambda i: (0, 0)),
            pl.BlockSpec((1, _D), lambda i: (0, 0)),
            pl.BlockSpec((1, _D), lambda i: (0, 0)),
        ],
        out_specs=pl.BlockSpec((rows, _D), lambda i: (i, 0)),
        out_shape=jax.ShapeDtypeStruct((_B * _TOPK, _D), jnp.float32),
        compiler_params=pltpu.CompilerParams(
            dimension_semantics=("arbitrary",),
        ),
    )(embs_flat, pos_emb, ln_gamma, ln_beta)


def kernel(encoder_hidden_states, preds_attr, word_emb, pos_emb, ln_gamma, ln_beta):
    labels = _make_sc_topk()(preds_attr)[:, :_TOPK]
    idx_flat = labels.reshape(_B * _TOPK)
    embs_flat = _make_sc_gather()(word_emb, idx_flat)
    out = _ln(
        embs_flat,
        pos_emb,
        ln_gamma.reshape(1, _D),
        ln_beta.reshape(1, _D),
    )
    return out.reshape(_B, _TOPK, _D), labels


# speedup vs baseline: 6.1121x; 2.6487x over previous
import functools

import jax
import jax.numpy as jnp
from jax import lax
from jax.experimental import pallas as pl
from jax.experimental.pallas import tpu as pltpu
from jax.experimental.pallas import tpu_sc as plsc

_B = 1024
_K = 100000
_TOPK = 30
_D = 128
_EPS = 1e-12

_NC, _NS, _L = 2, 16, 16
_NW = _NC * _NS
_RPW = _B // _NW
_NV = _K // _L
_HALF = _K // 2
_HV = _HALF // _L
_AU = 5
_BU = 10
_CAP = 4096
_NEG = float("-inf")
_POS = float("inf")
_IMAX = 2**31 - 1


def _extract30(val_at, idx_at, nv, iota):
    lab_a = jnp.zeros((_L,), jnp.int32)
    lab_b = jnp.zeros((_L,), jnp.int32)
    mm = jnp.full((_L,), _POS, jnp.float32)
    ps = jnp.full((_L,), -1, jnp.int32)
    for k in range(_TOPK):
        def scan(j, mp):
            m, p = mp
            v = val_at(j)
            ix = idx_at(j)
            elig = (v < mm) | ((v == mm) & (ix > ps))
            veff = jnp.where(elig, v, _NEG)
            gt = veff > m
            return jnp.where(gt, veff, m), jnp.where(gt, ix, p)

        m, p = lax.fori_loop(
            0, nv, scan,
            (jnp.full((_L,), _NEG, jnp.float32), jnp.zeros((_L,), jnp.int32)),
        )
        mm_s = jnp.max(m)
        ps_s = jnp.min(jnp.where(m == mm_s, p, _IMAX))
        if k < _L:
            lab_a = jnp.where(iota == k, ps_s, lab_a)
        else:
            lab_b = jnp.where(iota == (k - _L), ps_s, lab_b)
        mm = jnp.full((_L,), mm_s, jnp.float32)
        ps = jnp.full((_L,), ps_s, jnp.int32)
    return lab_a, lab_b


def _make_sc_topk():
    mesh = plsc.VectorSubcoreMesh(core_axis_name="c", subcore_axis_name="s")

    @functools.partial(
        pl.kernel,
        mesh=mesh,
        out_type=jax.ShapeDtypeStruct((_B, 32), jnp.int32),
        scratch_types=[
            pltpu.VMEM((_K,), jnp.float32),
            pltpu.VMEM((_CAP + 16,), jnp.float32),
            pltpu.VMEM((_CAP + 16,), jnp.int32),
            pltpu.VMEM((32,), jnp.int32),
            pltpu.SemaphoreType.DMA,
        ],
        compiler_params=pltpu.CompilerParams(needs_layout_passes=False),
    )
    def topk_k(preds_hbm, out_hbm, row_v, candv, candi, lab_v, sem):
        wid = lax.axis_index("s") * _NC + lax.axis_index("c")
        iota = lax.broadcasted_iota(jnp.int32, (_L,), 0)
        r0 = wid * _RPW
        pltpu.async_copy(
            preds_hbm.at[pl.ds(r0 * _K, _HALF)], row_v.at[pl.ds(0, _HALF)], sem)
        pltpu.async_copy(
            preds_hbm.at[pl.ds(r0 * _K + _HALF, _HALF)],
            row_v.at[pl.ds(_HALF, _HALF)], sem)

        def row_body(i, _):
            r = r0 + i

            def pass_a(base):
                def stp(jo, m12):
                    m1, m2 = m12
                    for u in range(_AU):
                        v = row_v[pl.ds(base + (jo * _AU + u) * _L, _L)]
                        gt1 = v > m1
                        m2 = jnp.where(gt1, m1, jnp.maximum(m2, v))
                        m1 = jnp.maximum(m1, v)
                    return m1, m2
                return stp

            neg = jnp.full((_L,), _NEG, jnp.float32)
            pltpu.make_async_copy(
                preds_hbm.at[pl.ds(r * _K, _HALF)],
                row_v.at[pl.ds(0, _HALF)], sem).wait()
            m12 = lax.fori_loop(0, _HV // _AU, pass_a(0), (neg, neg))
            pltpu.make_async_copy(
                preds_hbm.at[pl.ds(r * _K + _HALF, _HALF)],
                row_v.at[pl.ds(_HALF, _HALF)], sem).wait()
            _, m2 = lax.fori_loop(0, _HV // _AU, pass_a(_HALF), m12)
            t_vec = jnp.full((_L,), jnp.min(m2), jnp.float32)

            def pass_b(jo, cnt):
                vs, masks = [], []
                grp_any = None
                for u in range(_BU):
                    v = row_v[pl.ds((jo * _BU + u) * _L, _L)]
                    m = v >= t_vec
                    vs.append(v)
                    masks.append(m)
                    grp_any = m if grp_any is None else (grp_any | m)

                def append(c):
                    for u in range(_BU):
                        base = jnp.minimum(c, _CAP)
                        plsc.store_compressed(
                            candv.at[pl.ds(base, _L)], vs[u], mask=masks[u])
                        plsc.store_compressed(
                            candi.at[pl.ds(base, _L)],
                            iota + (jo * _BU + u) * _L, mask=masks[u])
                        c = c + jnp.sum(masks[u].astype(jnp.int32))
                    return c

                return lax.cond(jnp.any(grp_any), append, lambda c: c, cnt)

            cnt = lax.fori_loop(0, _NV // _BU, pass_b, jnp.int32(0))
            candv[pl.ds(jnp.minimum(cnt, _CAP), _L)] = neg

            def fast(_):
                return _extract30(
                    lambda j: candv[pl.ds(j * _L, _L)],
                    lambda j: candi[pl.ds(j * _L, _L)],
                    (cnt + _L - 1) // _L, iota)

            def slow(_):
                return _extract30(
                    lambda j: row_v[pl.ds(j * _L, _L)],
                    lambda j: iota + j * _L,
                    _NV, iota)

            lab_a, lab_b = lax.cond(cnt <= _CAP, fast, slow, 0)
            lab_v[pl.ds(0, _L)] = lab_a
            lab_v[pl.ds(_L, _L)] = lab_b
            pltpu.sync_copy(lab_v, out_hbm.at[r])

            @pl.when(i + 1 < _RPW)
            def _():
                pltpu.async_copy(
                    preds_hbm.at[pl.ds((r + 1) * _K, _HALF)],
                    row_v.at[pl.ds(0, _HALF)], sem)
                pltpu.async_copy(
                    preds_hbm.at[pl.ds((r + 1) * _K + _HALF, _HALF)],
                    row_v.at[pl.ds(_HALF, _HALF)], sem)
            return 0

        lax.fori_loop(0, _RPW, row_body, 0)

    return topk_k


def _make_sc_gather():
    n = _B * _TOPK
    b_per_w = n // _NW
    mesh = plsc.VectorSubcoreMesh(core_axis_name="c", subcore_axis_name="s")

    @functools.partial(
        pl.kernel,
        mesh=mesh,
        out_type=jax.ShapeDtypeStruct((n, _D), jnp.float32),
        scratch_types=[
            pltpu.VMEM((b_per_w,), jnp.int32),
            pltpu.VMEM((b_per_w, _D), jnp.float32),
            pltpu.SemaphoreType.DMA,
        ],
    )
    def gather_k(table_hbm, idx_hbm, out_hbm, idx_v, rows_v, sem):
        wid = lax.axis_index("s") * _NC + lax.axis_index("c")
        base = wid * b_per_w
        pltpu.sync_copy(idx_hbm.at[pl.ds(base, b_per_w)], idx_v)
        pltpu.async_copy(table_hbm.at[idx_v], rows_v, sem).wait()
        pltpu.sync_copy(rows_v, out_hbm.at[pl.ds(base, b_per_w)])

    return gather_k


def _ln_body(x_ref, pos_ref, g_ref, b_ref, o_ref):
    x = x_ref[...]
    pos = jnp.tile(pos_ref[...], (x.shape[0] // _TOPK, 1))
    y = x + pos
    mu = jnp.mean(y, axis=1, keepdims=True)
    d = y - mu
    var = jnp.mean(d * d, axis=1, keepdims=True)
    o_ref[...] = d / jnp.sqrt(var + _EPS) * g_ref[...] + b_ref[...]


def _ln(embs_flat, pos_emb, ln_gamma, ln_beta):
    rows = 240
    grid = (_B * _TOPK) // rows
    return pl.pallas_call(
        _ln_body,
        grid=(grid,),
        in_specs=[
            pl.BlockSpec((rows, _D), lambda i: (i, 0)),
            pl.BlockSpec((_TOPK, _D), lambda i: (0, 0)),
            pl.BlockSpec((1, _D), lambda i: (0, 0)),
            pl.BlockSpec((1, _D), lambda i: (0, 0)),
        ],
        out_specs=pl.BlockSpec((rows, _D), lambda i: (i, 0)),
        out_shape=jax.ShapeDtypeStruct((_B * _TOPK, _D), jnp.float32),
        compiler_params=pltpu.CompilerParams(
            dimension_semantics=("arbitrary",),
        ),
    )(embs_flat, pos_emb, ln_gamma, ln_beta)


def kernel(encoder_hidden_states, preds_attr, word_emb, pos_emb, ln_gamma, ln_beta):
    labels = _make_sc_topk()(preds_attr.reshape(_B * _K))[:, :_TOPK]
    idx_flat = labels.reshape(_B * _TOPK)
    embs_flat = _make_sc_gather()(word_emb, idx_flat)
    out = _ln(
        embs_flat,
        pos_emb,
        ln_gamma.reshape(1, _D),
        ln_beta.reshape(1, _D),
    )
    return out.reshape(_B, _TOPK, _D), labels
